# Initial kernel scaffold; baseline (speedup 1.0000x reference)
import jax
import jax.numpy as jnp
from jax.experimental import pallas as pl


def kernel(entity_emb, relation_emb, item_bias, entity_pairs, entity2edges, edge2entities, edge2relation):
    raise NotImplementedError("write your pallas kernel here")


# speedup vs baseline: 1.6177x; 1.6177x over previous
import jax
import jax.numpy as jnp
from jax import lax
from jax.experimental import pallas as pl
from jax.experimental.pallas import tpu as pltpu
from jax.experimental.pallas import tpu_sc as plsc

DIM = 128
NS = 32
BS = 4096
NSLOTS = 2 * BS
NWORKERS = 32
SLOTS_PER_W = NSLOTS // NWORKERS
CHUNK = 128
NE = NS * CHUNK
PAIRS_PER_W = SLOTS_PER_W // 2
L = 16


def _sc_body(emb, rel, bias, pairs, e2edges, e2ents, e2rel, out,
             eids_v, eidx_v, edgesT_v, n0i_v, n1i_v, n0T_v, n1T_v, relT_v,
             accE_v, accR_v, selfv_v, biasv_v, prod_v, outv_v, semI, semA):
    wid = lax.axis_index("s") * 2 + lax.axis_index("c")
    slot_base = wid * SLOTS_PER_W

    pltpu.sync_copy(pairs.at[pl.ds(slot_base, SLOTS_PER_W)], eids_v)
    pltpu.async_copy(bias.at[eids_v], biasv_v, semI).wait()

    for c in range(SLOTS_PER_W // CHUNK):
        def _bi_s(s, _):
            def _bi_g(g, _):
                ev = eids_v[pl.ds(c * CHUNK + g * L, L)]
                eidx_v[pl.ds(s * CHUNK + g * L, L)] = ev * NS + s
                return 0
            return lax.fori_loop(0, CHUNK // L, _bi_g, 0)
        lax.fori_loop(0, NS, _bi_s, 0)

        pltpu.async_copy(e2edges.at[eidx_v], edgesT_v, semI).wait()

        rel_cp = pltpu.async_copy(e2rel.at[edgesT_v], relT_v, semI)

        def _ni(i, _):
            v = edgesT_v[pl.ds(i * L, L)]
            n0i_v[pl.ds(i * L, L)] = 2 * v
            n1i_v[pl.ds(i * L, L)] = 2 * v + 1
            return 0
        lax.fori_loop(0, NE // L, _ni, 0)

        n0_cp = pltpu.async_copy(e2ents.at[n0i_v], n0T_v, semI)
        n1_cp = pltpu.async_copy(e2ents.at[n1i_v], n1T_v, semI)
        rel_cp.wait()
        n0_cp.wait()
        n1_cp.wait()

        initE = pltpu.async_copy(emb.at[n0T_v.at[pl.ds(0, CHUNK)]],
                                 accE_v, semI)
        initR = pltpu.async_copy(rel.at[relT_v.at[pl.ds(0, CHUNK)]],
                                 accR_v, semI)
        initE.wait()
        initR.wait()

        pltpu.async_copy(emb.at[eids_v.at[pl.ds(c * CHUNK, CHUNK)]],
                         selfv_v, semA)
        pltpu.async_copy(emb.at[n1T_v.at[pl.ds(0, CHUNK)]], accE_v, semA,
                         add=True)

        def _acc_s(s, _):
            sl = pl.ds(s * CHUNK, CHUNK)
            pltpu.async_copy(emb.at[n0T_v.at[sl]], accE_v, semA, add=True)
            pltpu.async_copy(emb.at[n1T_v.at[sl]], accE_v, semA, add=True)
            pltpu.async_copy(rel.at[relT_v.at[sl]], accR_v, semA, add=True)
            return 0
        lax.fori_loop(1, NS, _acc_s, 0)

        def _drain(i, _):
            pltpu.make_async_copy(emb.at[pl.ds(0, CHUNK)], accE_v,
                                  semA).wait()
            return 0
        lax.fori_loop(0, 2 + 3 * (NS - 1), _drain, 0)

        def _dot(p, _):
            j0 = 2 * p
            j1 = 2 * p + 1
            acc = jnp.zeros((L,), jnp.float32)
            for g in range(DIM // L):
                sl = pl.ds(g * L, L)
                v0 = (selfv_v[j0, sl] + accE_v[j0, sl] * (1.0 / 64.0)
                      + accR_v[j0, sl] * (1.0 / 32.0))
                v1 = (selfv_v[j1, sl] + accE_v[j1, sl] * (1.0 / 64.0)
                      + accR_v[j1, sl] * (1.0 / 32.0))
                acc = acc + v0 * v1
            prod_v[pl.ds(p * L, L)] = acc
            return 0
        lax.fori_loop(0, CHUNK // 2, _dot, 0)

        def _hsum(g, _):
            rows = g * L + lax.iota(jnp.int32, L)
            acc = jnp.zeros((L,), jnp.float32)
            for k in range(L):
                acc = acc + plsc.load_gather(prod_v, [rows * L + k])
            outv_v[pl.ds(c * (CHUNK // 2) + g * L, L)] = acc
            return 0
        lax.fori_loop(0, CHUNK // 2 // L, _hsum, 0)

    def _bias_g(g, _):
        lanes = g * L + lax.iota(jnp.int32, L)
        b16 = plsc.load_gather(biasv_v, [2 * lanes + 1])
        outv_v[pl.ds(g * L, L)] = outv_v[pl.ds(g * L, L)] + b16
        return 0
    lax.fori_loop(0, PAIRS_PER_W // L, _bias_g, 0)
    pltpu.sync_copy(outv_v, out.at[pl.ds(wid * PAIRS_PER_W, PAIRS_PER_W)])


@jax.jit
def _run(emb, rel, bias, pairs_flat, e2edges_flat, e2ents_flat, e2rel):
    mesh = plsc.VectorSubcoreMesh(core_axis_name="c", subcore_axis_name="s")
    return pl.kernel(
        _sc_body,
        out_type=jax.ShapeDtypeStruct((BS,), jnp.float32),
        mesh=mesh,
        compiler_params=pltpu.CompilerParams(needs_layout_passes=False),
        scratch_types=[
            pltpu.VMEM((SLOTS_PER_W,), jnp.int32),
            pltpu.VMEM((NE,), jnp.int32),
            pltpu.VMEM((NE,), jnp.int32),
            pltpu.VMEM((NE,), jnp.int32),
            pltpu.VMEM((NE,), jnp.int32),
            pltpu.VMEM((NE,), jnp.int32),
            pltpu.VMEM((NE,), jnp.int32),
            pltpu.VMEM((NE,), jnp.int32),
            pltpu.VMEM((CHUNK, DIM), jnp.float32),
            pltpu.VMEM((CHUNK, DIM), jnp.float32),
            pltpu.VMEM((CHUNK, DIM), jnp.float32),
            pltpu.VMEM((SLOTS_PER_W,), jnp.float32),
            pltpu.VMEM((CHUNK // 2 * L,), jnp.float32),
            pltpu.VMEM((PAIRS_PER_W,), jnp.float32),
            pltpu.SemaphoreType.DMA,
            pltpu.SemaphoreType.DMA,
        ],
    )(emb, rel, bias, pairs_flat, e2edges_flat, e2ents_flat, e2rel)


def kernel(entity_emb, relation_emb, item_bias, entity_pairs, entity2edges,
           edge2entities, edge2relation):
    pairs_flat = jnp.asarray(entity_pairs, jnp.int32).reshape(NSLOTS)
    return _run(entity_emb, relation_emb, item_bias, pairs_flat,
                entity2edges.reshape(-1), edge2entities.reshape(-1),
                edge2relation)
